# baseline (speedup 1.0000x reference)
import functools

import jax
import jax.numpy as jnp
from jax import lax
from jax.experimental import pallas as pl
from jax.experimental.pallas import tpu as pltpu
from jax.experimental.pallas import tpu_sc as plsc

NUM_HEADS = 16
VOCAB = 512
B_TOTAL = 8 * 512 * 512
NW = 32
CHUNK = 1024
N_CHUNKS = B_TOTAL // CHUNK
CPW = N_CHUNKS // NW

_mesh = plsc.VectorSubcoreMesh(core_axis_name="c", subcore_axis_name="s")


@functools.partial(
    pl.kernel,
    mesh=_mesh,
    out_type=jax.ShapeDtypeStruct((8, 512, 2, 4, 8, 128), jnp.float32),
    scratch_types=[
        pltpu.VMEM((2, CHUNK), jnp.int32),
        pltpu.VMEM((2, CHUNK, NUM_HEADS), jnp.float32),
        pltpu.VMEM((2, 2, 4, 2, 8, 129), jnp.float32),
        pltpu.VMEM_SHARED((VOCAB, NUM_HEADS), jnp.float32),
        pltpu.SemaphoreType.DMA((2,)),
        pltpu.SemaphoreType.DMA((2,)),
        pltpu.SemaphoreType.DMA((2,)),
    ],
    compiler_params=pltpu.CompilerParams(use_tc_tiling_on_sc=False,
                                         needs_layout_passes=False),
)
def _gather_kernel(table_hbm, idx_hbm, out_hbm, idx_v, rows_v, out_pad,
                   table_sh, idx_sem, gat_sem, wb_sem):
    sid = lax.axis_index("s")
    w = sid * 2 + lax.axis_index("c")
    c0 = w * CPW

    @pl.when(sid == 0)
    def _stage_table():
        pltpu.sync_copy(table_hbm, table_sh)

    plsc.subcore_barrier()

    iota16 = lax.broadcasted_iota(jnp.int32, (16,), 0)
    ht_vec = iota16 // 8
    hh_vec = iota16 % 8

    def decode(c):
        return c // 256, c % 256

    def start_idx(c, buf):
        pltpu.async_copy(idx_hbm.at[pl.ds(c * CHUNK, CHUNK)],
                         idx_v.at[buf], idx_sem.at[buf])

    def wait_idx(buf):
        pltpu.make_async_copy(idx_hbm.at[pl.ds(0, CHUNK)],
                              idx_v.at[buf], idx_sem.at[buf]).wait()

    def start_gathers(buf):
        pltpu.async_copy(table_sh.at[idx_v.at[buf]], rows_v.at[buf],
                         gat_sem.at[buf])

    def wait_gathers(buf):
        pltpu.make_async_copy(table_sh.at[idx_v.at[buf]], rows_v.at[buf],
                              gat_sem.at[buf]).wait()

    def start_wb(c, buf):
        b, i2 = decode(c)
        for il in range(2):
            for ht in range(2):
                pltpu.async_copy(
                    out_pad.at[buf, il, pl.ds(0, 4), ht, pl.ds(0, 8),
                               pl.ds(0, 128)],
                    out_hbm.at[b, i2 * 2 + il, ht],
                    wb_sem.at[buf])

    def wait_wb(buf):
        for il in range(2):
            for ht in range(2):
                pltpu.make_async_copy(
                    out_pad.at[buf, il, pl.ds(0, 4), ht, pl.ds(0, 8),
                               pl.ds(0, 128)],
                    out_hbm.at[0, 0, ht],
                    wb_sem.at[buf]).wait()

    def compute(buf):
        for il in range(2):
            for jt in range(4):
                tgt = out_pad.at[buf, il, jt]
                base = il * 512 + jt * 128

                @plsc.parallel_loop(0, 128, 1, unroll=16)
                def _body(jj, tgt=tgt, base=base):
                    col = jnp.full((16,), jj, jnp.int32)
                    vals = rows_v[buf, base + jj]
                    plsc.store_scatter(tgt, [ht_vec, hh_vec, col], vals)

    def run_chunk(c, buf, skip_wb_wait, has1, has2):
        if has1:
            wait_idx(1 - buf)
            start_gathers(1 - buf)
        wait_gathers(buf)
        if not skip_wb_wait:
            wait_wb(buf)
        compute(buf)
        start_wb(c, buf)
        if has2:
            start_idx(c + 2, buf)

    start_idx(c0, 0)
    start_idx(c0 + 1, 1)
    wait_idx(0)
    start_gathers(0)
    run_chunk(c0 + 0, 0, True, True, True)
    run_chunk(c0 + 1, 1, True, True, True)

    def rounds(r, carry):
        g = c0 + 2 + 2 * r
        run_chunk(g, 0, False, True, True)
        run_chunk(g + 1, 1, False, True, True)
        return carry

    lax.fori_loop(0, (CPW - 4) // 2, rounds, 0)

    run_chunk(c0 + CPW - 2, 0, False, True, False)
    run_chunk(c0 + CPW - 1, 1, False, False, False)
    wait_wb(0)
    wait_wb(1)


def kernel(dist, embedding_table):
    idx = dist.reshape(-1).astype(jnp.int32)
    out = _gather_kernel(embedding_table, idx)
    return out.transpose(0, 1, 3, 5, 2, 4).reshape(8, 512, 512, NUM_HEADS)


# speedup vs baseline: 1.0378x; 1.0378x over previous
import functools

import jax
import jax.numpy as jnp
from jax import lax
from jax.experimental import pallas as pl
from jax.experimental.pallas import tpu as pltpu
from jax.experimental.pallas import tpu_sc as plsc

NUM_HEADS = 16
VOCAB = 512
B_TOTAL = 8 * 512 * 512
NW = 32
CHUNK = 1024
N_CHUNKS = B_TOTAL // CHUNK
CPW = N_CHUNKS // NW

_mesh = plsc.VectorSubcoreMesh(core_axis_name="c", subcore_axis_name="s")


@functools.partial(
    pl.kernel,
    mesh=_mesh,
    out_type=jax.ShapeDtypeStruct((8, 512, 2, 4, 8, 128), jnp.float32),
    scratch_types=[
        pltpu.VMEM((2, CHUNK), jnp.int32),
        pltpu.VMEM((2, CHUNK, NUM_HEADS), jnp.float32),
        pltpu.VMEM((2, 128, 129), jnp.float32),
        pltpu.VMEM_SHARED((VOCAB, NUM_HEADS), jnp.float32),
        pltpu.SemaphoreType.DMA((2,)),
        pltpu.SemaphoreType.DMA((2,)),
        pltpu.SemaphoreType.DMA((2,)),
    ],
    compiler_params=pltpu.CompilerParams(use_tc_tiling_on_sc=False,
                                         needs_layout_passes=False),
)
def _gather_kernel(table_hbm, idx_hbm, out_hbm, idx_v, rows_v, out_pad,
                   table_sh, idx_sem, gat_sem, wb_sem):
    sid = lax.axis_index("s")
    w = sid * 2 + lax.axis_index("c")
    c0 = w * CPW

    @pl.when(sid == 0)
    def _stage_table():
        pltpu.sync_copy(table_hbm, table_sh)

    plsc.subcore_barrier()

    iota16 = lax.broadcasted_iota(jnp.int32, (16,), 0)
    row_ids = [jnp.full((16,), ii * 16, jnp.int32) + iota16 for ii in range(8)]

    def decode(c):
        return c // 256, (c % 256) // 4, c % 4

    def start_idx(c, buf):
        pltpu.async_copy(idx_hbm.at[pl.ds(c * CHUNK, CHUNK)],
                         idx_v.at[buf], idx_sem.at[buf])

    def wait_idx(buf):
        pltpu.make_async_copy(idx_hbm.at[pl.ds(0, CHUNK)],
                              idx_v.at[buf], idx_sem.at[buf]).wait()

    def start_gathers(buf):
        pltpu.async_copy(table_sh.at[idx_v.at[buf]], rows_v.at[buf],
                         gat_sem.at[buf])

    def wait_gathers(buf):
        pltpu.make_async_copy(table_sh.at[idx_v.at[buf]], rows_v.at[buf],
                              gat_sem.at[buf]).wait()

    def start_wb(c, buf):
        b, it, jt = decode(c)
        for ii in range(8):
            for ht in range(2):
                pltpu.async_copy(
                    out_pad.at[buf, pl.ds(ii * 16 + ht * 8, 8),
                               pl.ds(0, 128)],
                    out_hbm.at[b, it * 8 + ii, ht, jt],
                    wb_sem.at[buf])

    def wait_wb(buf):
        for ii in range(8):
            for ht in range(2):
                pltpu.make_async_copy(
                    out_pad.at[buf, pl.ds(ii * 16 + ht * 8, 8),
                               pl.ds(0, 128)],
                    out_hbm.at[0, 0, ht, 0],
                    wb_sem.at[buf]).wait()

    def compute(buf):
        @plsc.parallel_loop(0, 128, 1, unroll=8)
        def _body(jj):
            col = jnp.full((16,), jj, jnp.int32)
            for ii in range(8):
                vals = rows_v[buf, ii * 128 + jj]
                plsc.store_scatter(out_pad.at[buf], [row_ids[ii], col], vals)

    def run_chunk(c, buf, skip_wb_wait, has1, has2):
        if has1:
            wait_idx(1 - buf)
            start_gathers(1 - buf)
        wait_gathers(buf)
        if not skip_wb_wait:
            wait_wb(buf)
        compute(buf)
        start_wb(c, buf)
        if has2:
            start_idx(c + 2, buf)

    start_idx(c0, 0)
    start_idx(c0 + 1, 1)
    wait_idx(0)
    start_gathers(0)
    run_chunk(c0 + 0, 0, True, True, True)
    run_chunk(c0 + 1, 1, True, True, True)

    def rounds(r, carry):
        g = c0 + 2 + 2 * r
        run_chunk(g, 0, False, True, True)
        run_chunk(g + 1, 1, False, True, True)
        return carry

    lax.fori_loop(0, (CPW - 4) // 2, rounds, 0)

    run_chunk(c0 + CPW - 2, 0, False, True, False)
    run_chunk(c0 + CPW - 1, 1, False, False, False)
    wait_wb(0)
    wait_wb(1)


def kernel(dist, embedding_table):
    idx = (dist.astype(jnp.int32)
           .reshape(8, 64, 8, 4, 128)
           .transpose(0, 1, 3, 2, 4)
           .reshape(B_TOTAL))
    out = _gather_kernel(embedding_table, idx)
    return out.transpose(0, 1, 3, 5, 2, 4).reshape(8, 512, 512, NUM_HEADS)


# speedup vs baseline: 1.1851x; 1.1420x over previous
import functools

import jax
import jax.numpy as jnp
from jax import lax
from jax.experimental import pallas as pl
from jax.experimental.pallas import tpu as pltpu
from jax.experimental.pallas import tpu_sc as plsc

NUM_HEADS = 16
VOCAB = 512
B_TOTAL = 8 * 512 * 512
NW = 32
CHUNK = 1024
N_CHUNKS = B_TOTAL // CHUNK
CPW = N_CHUNKS // NW

_mesh = plsc.VectorSubcoreMesh(core_axis_name="c", subcore_axis_name="s")


@functools.partial(
    pl.kernel,
    mesh=_mesh,
    out_type=jax.ShapeDtypeStruct((8, 512, 2, 4, 8, 128), jnp.float32),
    scratch_types=[
        pltpu.VMEM((2, CHUNK), jnp.int32),
        pltpu.VMEM((2, CHUNK, NUM_HEADS), jnp.float32),
        pltpu.VMEM((2, 128, 129), jnp.float32),
        pltpu.VMEM_SHARED((VOCAB, NUM_HEADS), jnp.float32),
        pltpu.SemaphoreType.DMA((2,)),
        pltpu.SemaphoreType.DMA((2,)),
        pltpu.SemaphoreType.DMA((2,)),
    ],
    compiler_params=pltpu.CompilerParams(use_tc_tiling_on_sc=False,
                                         needs_layout_passes=False),
)
def _gather_kernel(table_hbm, idx_hbm, out_hbm, idx_v, rows_v, out_pad,
                   table_sh, idx_sem, gat_sem, wb_sem):
    sid = lax.axis_index("s")
    w = sid * 2 + lax.axis_index("c")
    c0 = w * CPW

    @pl.when(sid == 0)
    def _stage_table():
        pltpu.sync_copy(table_hbm, table_sh)

    plsc.subcore_barrier()

    iota16 = lax.broadcasted_iota(jnp.int32, (16,), 0)
    row_ids = [jnp.full((16,), ii * 16, jnp.int32) + iota16 for ii in range(8)]

    def decode(c):
        return c // 256, (c % 256) // 4, c % 4

    def start_idx(c, buf):
        pltpu.async_copy(idx_hbm.at[pl.ds(c * CHUNK, CHUNK)],
                         idx_v.at[buf], idx_sem.at[buf])

    def wait_idx(buf):
        pltpu.make_async_copy(idx_hbm.at[pl.ds(0, CHUNK)],
                              idx_v.at[buf], idx_sem.at[buf]).wait()

    def start_gathers(buf):
        pltpu.async_copy(table_sh.at[idx_v.at[buf]], rows_v.at[buf],
                         gat_sem.at[buf])

    def wait_gathers(buf):
        pltpu.make_async_copy(table_sh.at[idx_v.at[buf]], rows_v.at[buf],
                              gat_sem.at[buf]).wait()

    def start_wb(c, buf):
        b, it, jt = decode(c)
        for ii in range(8):
            for ht in range(2):
                pltpu.async_copy(
                    out_pad.at[buf, pl.ds(ii * 16 + ht * 8, 8),
                               pl.ds(0, 128)],
                    out_hbm.at[b, it * 8 + ii, ht, jt],
                    wb_sem.at[buf])

    def wait_wb(buf):
        for ii in range(8):
            for ht in range(2):
                pltpu.make_async_copy(
                    out_pad.at[buf, pl.ds(ii * 16 + ht * 8, 8),
                               pl.ds(0, 128)],
                    out_hbm.at[0, 0, ht, 0],
                    wb_sem.at[buf]).wait()

    def compute(buf):
        @plsc.parallel_loop(0, 128, 1, unroll=2)
        def _body(jj):
            col = jnp.full((16,), jj, jnp.int32)
            for ii in range(8):
                vals = rows_v[buf, ii * 128 + jj]
                plsc.store_scatter(out_pad.at[buf], [row_ids[ii], col], vals)

    def run_chunk(c, buf, skip_wb_wait, has1, has2):
        if has1:
            wait_idx(1 - buf)
            start_gathers(1 - buf)
        wait_gathers(buf)
        if not skip_wb_wait:
            wait_wb(buf)
        compute(buf)
        start_wb(c, buf)
        if has2:
            start_idx(c + 2, buf)

    start_idx(c0, 0)
    start_idx(c0 + 1, 1)
    wait_idx(0)
    start_gathers(0)
    run_chunk(c0 + 0, 0, True, True, True)
    run_chunk(c0 + 1, 1, True, True, True)

    def rounds(r, carry):
        g = c0 + 2 + 2 * r
        run_chunk(g, 0, False, True, True)
        run_chunk(g + 1, 1, False, True, True)
        return carry

    lax.fori_loop(0, (CPW - 4) // 2, rounds, 0)

    run_chunk(c0 + CPW - 2, 0, False, True, False)
    run_chunk(c0 + CPW - 1, 1, False, False, False)
    wait_wb(0)
    wait_wb(1)


def kernel(dist, embedding_table):
    idx = (dist.astype(jnp.int32)
           .reshape(8, 64, 8, 4, 128)
           .transpose(0, 1, 3, 2, 4)
           .reshape(B_TOTAL))
    out = _gather_kernel(embedding_table, idx)
    return out.transpose(0, 1, 3, 5, 2, 4).reshape(8, 512, 512, NUM_HEADS)


# speedup vs baseline: 1.1856x; 1.0004x over previous
import functools

import jax
import jax.numpy as jnp
from jax import lax
from jax.experimental import pallas as pl
from jax.experimental.pallas import tpu as pltpu
from jax.experimental.pallas import tpu_sc as plsc

NUM_HEADS = 16
VOCAB = 512
B_TOTAL = 8 * 512 * 512
NW = 32
CHUNK = 1024
N_CHUNKS = B_TOTAL // CHUNK
CPW = N_CHUNKS // NW

_mesh = plsc.VectorSubcoreMesh(core_axis_name="c", subcore_axis_name="s")


@functools.partial(
    pl.kernel,
    mesh=_mesh,
    out_type=jax.ShapeDtypeStruct((8, 512, 2, 4, 8, 128), jnp.float32),
    scratch_types=[
        pltpu.VMEM((2, CHUNK), jnp.int32),
        pltpu.VMEM((2, CHUNK, NUM_HEADS), jnp.float32),
        pltpu.VMEM((2, 128, 129), jnp.float32),
        pltpu.VMEM_SHARED((VOCAB, NUM_HEADS), jnp.float32),
        pltpu.SemaphoreType.DMA((2,)),
        pltpu.SemaphoreType.DMA((2,)),
        pltpu.SemaphoreType.DMA((2,)),
    ],
    compiler_params=pltpu.CompilerParams(use_tc_tiling_on_sc=False,
                                         needs_layout_passes=False),
)
def _gather_kernel(table_hbm, idx_hbm, out_hbm, idx_v, rows_v, out_pad,
                   table_sh, idx_sem, gat_sem, wb_sem):
    sid = lax.axis_index("s")
    w = sid * 2 + lax.axis_index("c")
    c0 = w * CPW

    @pl.when(sid == 0)
    def _stage_table():
        pltpu.sync_copy(table_hbm, table_sh)

    plsc.subcore_barrier()

    iota16 = lax.broadcasted_iota(jnp.int32, (16,), 0)
    row_ids = [jnp.full((16,), ii * 16, jnp.int32) + iota16 for ii in range(8)]

    def decode(c):
        return c // 256, (c % 256) // 4, c % 4

    def start_idx(c, buf):
        pltpu.async_copy(idx_hbm.at[pl.ds(c * CHUNK, CHUNK)],
                         idx_v.at[buf], idx_sem.at[buf])

    def wait_idx(buf):
        pltpu.make_async_copy(idx_hbm.at[pl.ds(0, CHUNK)],
                              idx_v.at[buf], idx_sem.at[buf]).wait()

    def start_gathers(buf):
        pltpu.async_copy(table_sh.at[idx_v.at[buf]], rows_v.at[buf],
                         gat_sem.at[buf])

    def wait_gathers(buf):
        pltpu.make_async_copy(table_sh.at[idx_v.at[buf]], rows_v.at[buf],
                              gat_sem.at[buf]).wait()

    def start_wb(c, buf):
        b, it, jt = decode(c)
        for ii in range(8):
            for ht in range(2):
                pltpu.async_copy(
                    out_pad.at[buf, pl.ds(ii * 16 + ht * 8, 8),
                               pl.ds(0, 128)],
                    out_hbm.at[b, it * 8 + ii, ht, jt],
                    wb_sem.at[buf])

    def wait_wb(buf):
        for ii in range(8):
            for ht in range(2):
                pltpu.make_async_copy(
                    out_pad.at[buf, pl.ds(ii * 16 + ht * 8, 8),
                               pl.ds(0, 128)],
                    out_hbm.at[0, 0, ht, 0],
                    wb_sem.at[buf]).wait()

    def compute(buf):
        @plsc.parallel_loop(0, 128, 1, unroll=4)
        def _body(jj):
            col = jnp.full((16,), jj, jnp.int32)
            for ii in range(8):
                vals = rows_v[buf, ii * 128 + jj]
                plsc.store_scatter(out_pad.at[buf], [row_ids[ii], col], vals)

    def run_chunk(c, buf, skip_wb_wait, has1, has2):
        if has1:
            wait_idx(1 - buf)
            start_gathers(1 - buf)
        wait_gathers(buf)
        if not skip_wb_wait:
            wait_wb(buf)
        compute(buf)
        start_wb(c, buf)
        if has2:
            start_idx(c + 2, buf)

    start_idx(c0, 0)
    start_idx(c0 + 1, 1)
    wait_idx(0)
    start_gathers(0)
    run_chunk(c0 + 0, 0, True, True, True)
    run_chunk(c0 + 1, 1, True, True, True)

    def rounds(r, carry):
        g = c0 + 2 + 2 * r
        run_chunk(g, 0, False, True, True)
        run_chunk(g + 1, 1, False, True, True)
        return carry

    lax.fori_loop(0, (CPW - 4) // 2, rounds, 0)

    run_chunk(c0 + CPW - 2, 0, False, True, False)
    run_chunk(c0 + CPW - 1, 1, False, False, False)
    wait_wb(0)
    wait_wb(1)


def kernel(dist, embedding_table):
    idx = (dist.astype(jnp.int32)
           .reshape(8, 64, 8, 4, 128)
           .transpose(0, 1, 3, 2, 4)
           .reshape(B_TOTAL))
    out = _gather_kernel(embedding_table, idx)
    return out.transpose(0, 1, 3, 5, 2, 4).reshape(8, 512, 512, NUM_HEADS)


# speedup vs baseline: 1.7100x; 1.4424x over previous
import functools

import jax
import jax.numpy as jnp
from jax import lax
from jax.experimental import pallas as pl
from jax.experimental.pallas import tpu as pltpu
from jax.experimental.pallas import tpu_sc as plsc

NUM_HEADS = 16
VOCAB = 512
B_TOTAL = 8 * 512 * 512
NW = 32
CHUNK = 1024
N_CHUNKS = B_TOTAL // CHUNK
CPW = N_CHUNKS // NW

_mesh = plsc.VectorSubcoreMesh(core_axis_name="c", subcore_axis_name="s")


@functools.partial(
    pl.kernel,
    mesh=_mesh,
    out_type=jax.ShapeDtypeStruct((8, 512, 2, 4, 8, 128), jnp.float32),
    scratch_types=[
        pltpu.VMEM((3, CHUNK), jnp.int32),
        pltpu.VMEM((3, CHUNK, NUM_HEADS), jnp.float32),
        pltpu.VMEM((3, 128, 129), jnp.float32),
        pltpu.VMEM_SHARED((VOCAB, NUM_HEADS), jnp.float32),
        pltpu.SemaphoreType.DMA((3,)),
        pltpu.SemaphoreType.DMA((3,)),
        pltpu.SemaphoreType.DMA((3,)),
    ],
    compiler_params=pltpu.CompilerParams(use_tc_tiling_on_sc=False,
                                         needs_layout_passes=False),
)
def _gather_kernel(table_hbm, idx_hbm, out_hbm, idx_v, rows_v, out_pad,
                   table_sh, idx_sem, gat_sem, wb_sem):
    sid = lax.axis_index("s")
    w = sid * 2 + lax.axis_index("c")
    c0 = w * CPW

    @pl.when(sid == 0)
    def _stage_table():
        pltpu.sync_copy(table_hbm, table_sh)

    plsc.subcore_barrier()

    iota16 = lax.broadcasted_iota(jnp.int32, (16,), 0)
    row_ids = [jnp.full((16,), ii * 16, jnp.int32) + iota16 for ii in range(8)]

    def decode(c):
        return c // 256, (c % 256) // 4, c % 4

    def start_idx(c, buf):
        pltpu.async_copy(idx_hbm.at[pl.ds(c * CHUNK, CHUNK)],
                         idx_v.at[buf], idx_sem.at[buf])

    def wait_idx(buf):
        pltpu.make_async_copy(idx_hbm.at[pl.ds(0, CHUNK)],
                              idx_v.at[buf], idx_sem.at[buf]).wait()

    def start_gathers(buf):
        pltpu.async_copy(table_sh.at[idx_v.at[buf]], rows_v.at[buf],
                         gat_sem.at[buf])

    def wait_gathers(buf):
        pltpu.make_async_copy(table_sh.at[idx_v.at[buf]], rows_v.at[buf],
                              gat_sem.at[buf]).wait()

    def start_wb(c, buf):
        b, it, jt = decode(c)
        for ii in range(8):
            for ht in range(2):
                pltpu.async_copy(
                    out_pad.at[buf, pl.ds(ii * 16 + ht * 8, 8),
                               pl.ds(0, 128)],
                    out_hbm.at[b, it * 8 + ii, ht, jt],
                    wb_sem.at[buf])

    def wait_wb(buf):
        for ii in range(8):
            for ht in range(2):
                pltpu.make_async_copy(
                    out_pad.at[buf, pl.ds(ii * 16 + ht * 8, 8),
                               pl.ds(0, 128)],
                    out_hbm.at[0, 0, ht, 0],
                    wb_sem.at[buf]).wait()

    def compute(buf):
        @plsc.parallel_loop(0, 128, 1, unroll=4)
        def _body(jj):
            col = jnp.full((16,), jj, jnp.int32)
            for ii in range(8):
                vals = rows_v[buf, ii * 128 + jj]
                plsc.store_scatter(out_pad.at[buf], [row_ids[ii], col], vals)

    def run_chunk(c, buf, skip_wb_wait, has1, has3):
        if has1:
            wait_idx((buf + 1) % 3)
            start_gathers((buf + 1) % 3)
        wait_gathers(buf)
        if not skip_wb_wait:
            wait_wb(buf)
        compute(buf)
        start_wb(c, buf)
        if has3:
            start_idx(c + 3, buf)

    for k in range(3):
        start_idx(c0 + k, k)
    wait_idx(0)
    start_gathers(0)
    for k in range(3):
        run_chunk(c0 + k, k, True, True, True)

    def rounds(r, carry):
        g = c0 + 3 + 3 * r
        for k in range(3):
            run_chunk(g + k, k, False, True, True)
        return carry

    lax.fori_loop(0, (CPW - 7) // 3, rounds, 0)

    for g in range(CPW - 4, CPW):
        run_chunk(c0 + g, g % 3, False, g + 1 < CPW, g + 3 < CPW)
    for k in range(3):
        wait_wb(k)


def kernel(dist, embedding_table):
    idx = (dist.astype(jnp.int32)
           .reshape(8, 64, 8, 4, 128)
           .transpose(0, 1, 3, 2, 4)
           .reshape(B_TOTAL))
    out = _gather_kernel(embedding_table, idx)
    return out.transpose(0, 1, 3, 5, 2, 4).reshape(8, 512, 512, NUM_HEADS)


# speedup vs baseline: 1.7205x; 1.0061x over previous
import functools

import jax
import jax.numpy as jnp
from jax import lax
from jax.experimental import pallas as pl
from jax.experimental.pallas import tpu as pltpu
from jax.experimental.pallas import tpu_sc as plsc

NUM_HEADS = 16
VOCAB = 512
B_TOTAL = 8 * 512 * 512
NW = 32
CHUNK = 512
N_CHUNKS = B_TOTAL // CHUNK
CPW = N_CHUNKS // NW
NBUF = 5

_mesh = plsc.VectorSubcoreMesh(core_axis_name="c", subcore_axis_name="s")


@functools.partial(
    pl.kernel,
    mesh=_mesh,
    out_type=jax.ShapeDtypeStruct((8, 512, 2, 4, 8, 128), jnp.float32),
    scratch_types=[
        pltpu.VMEM((NBUF, CHUNK), jnp.int32),
        pltpu.VMEM((NBUF, CHUNK, NUM_HEADS), jnp.float32),
        pltpu.VMEM((NBUF, 64, 129), jnp.float32),
        pltpu.VMEM_SHARED((VOCAB, NUM_HEADS), jnp.float32),
        pltpu.SemaphoreType.DMA((NBUF,)),
        pltpu.SemaphoreType.DMA((NBUF,)),
        pltpu.SemaphoreType.DMA((NBUF,)),
    ],
    compiler_params=pltpu.CompilerParams(use_tc_tiling_on_sc=False,
                                         needs_layout_passes=False),
)
def _gather_kernel(table_hbm, idx_hbm, out_hbm, idx_v, rows_v, out_pad,
                   table_sh, idx_sem, gat_sem, wb_sem):
    sid = lax.axis_index("s")
    w = sid * 2 + lax.axis_index("c")
    c0 = w * CPW

    @pl.when(sid == 0)
    def _stage_table():
        pltpu.sync_copy(table_hbm, table_sh)

    plsc.subcore_barrier()

    iota16 = lax.broadcasted_iota(jnp.int32, (16,), 0)
    row_ids = [jnp.full((16,), ii * 16, jnp.int32) + iota16 for ii in range(4)]

    def decode(c):
        return c // 512, (c % 512) // 8, (c % 8) // 2, c % 2

    def start_idx(c, buf):
        pltpu.async_copy(idx_hbm.at[pl.ds(c * CHUNK, CHUNK)],
                         idx_v.at[buf], idx_sem.at[buf])

    def wait_idx(buf):
        pltpu.make_async_copy(idx_hbm.at[pl.ds(0, CHUNK)],
                              idx_v.at[buf], idx_sem.at[buf]).wait()

    def start_gathers(buf):
        pltpu.async_copy(table_sh.at[idx_v.at[buf]], rows_v.at[buf],
                         gat_sem.at[buf])

    def wait_gathers(buf):
        pltpu.make_async_copy(table_sh.at[idx_v.at[buf]], rows_v.at[buf],
                              gat_sem.at[buf]).wait()

    def start_wb(c, buf):
        b, it, jt, half = decode(c)
        for ii in range(4):
            for ht in range(2):
                pltpu.async_copy(
                    out_pad.at[buf, pl.ds(ii * 16 + ht * 8, 8),
                               pl.ds(0, 128)],
                    out_hbm.at[b, it * 8 + half * 4 + ii, ht, jt],
                    wb_sem.at[buf])

    def wait_wb(buf):
        for ii in range(4):
            for ht in range(2):
                pltpu.make_async_copy(
                    out_pad.at[buf, pl.ds(ii * 16 + ht * 8, 8),
                               pl.ds(0, 128)],
                    out_hbm.at[0, 0, ht, 0],
                    wb_sem.at[buf]).wait()

    def compute(buf):
        @plsc.parallel_loop(0, 128, 1, unroll=4)
        def _body(jj):
            col = jnp.full((16,), jj, jnp.int32)
            for ii in range(4):
                vals = rows_v[buf, ii * 128 + jj]
                plsc.store_scatter(out_pad.at[buf], [row_ids[ii], col], vals)

    def run_chunk(c, buf, skip_wb_wait, has1, hasn):
        if has1:
            wait_idx((buf + 1) % NBUF)
            start_gathers((buf + 1) % NBUF)
        wait_gathers(buf)
        if not skip_wb_wait:
            wait_wb(buf)
        compute(buf)
        start_wb(c, buf)
        if hasn:
            start_idx(c + NBUF, buf)

    for k in range(NBUF):
        start_idx(c0 + k, k)
    wait_idx(0)
    start_gathers(0)
    for k in range(NBUF):
        run_chunk(c0 + k, k, True, True, True)

    n_steady = (CPW - 2 * NBUF) // NBUF

    def rounds(r, carry):
        g = c0 + NBUF + NBUF * r
        for k in range(NBUF):
            run_chunk(g + k, k, False, True, True)
        return carry

    lax.fori_loop(0, n_steady, rounds, 0)

    for g in range(NBUF + n_steady * NBUF, CPW):
        run_chunk(c0 + g, g % NBUF, False, g + 1 < CPW, g + NBUF < CPW)
    for k in range(NBUF):
        wait_wb(k)


def kernel(dist, embedding_table):
    idx = (dist.astype(jnp.int32)
           .reshape(8, 64, 8, 4, 128)
           .transpose(0, 1, 3, 2, 4)
           .reshape(B_TOTAL))
    out = _gather_kernel(embedding_table, idx)
    return out.transpose(0, 1, 3, 5, 2, 4).reshape(8, 512, 512, NUM_HEADS)
